# baseline (speedup 1.0000x reference)
import functools

import jax
import jax.numpy as jnp
from jax import lax
from jax.experimental import pallas as pl
from jax.experimental.pallas import tpu as pltpu
from jax.experimental.pallas import tpu_sc as plsc

N = 10000
D = 128
E = 320000
G = 64
OUT = 16

NC = 2
NS = 16
NW = NC * NS
EW = E // NW
C = 40
EWP = 10240
NCHUNK = EWP // C
RT = N // NS
RC = 25
NRC = RT // RC

NP = 10240
ET = E // NS
W16 = NP // NS


def _rsqrt16(d):
    i = plsc.bitcast(d, jnp.int32)
    y = plsc.bitcast(jnp.int32(0x5F3759DF) - (i >> 1), jnp.float32)
    for _ in range(3):
        y = y * (1.5 - 0.5 * d * y * y)
    return y


def _deg_body(dst_hbm, dinv_hbm, dstbuf, degbuf, tmpb, dinvb, repb, degsh):
    cid = lax.axis_index("c")
    sid = lax.axis_index("s")

    @pl.when(cid == 0)
    def _():
        def z(i, _):
            degbuf[pl.ds(i * 16, 16)] = jnp.zeros((16,), jnp.float32)
            return 0

        lax.fori_loop(0, NP // 16, z, 0)
        pltpu.sync_copy(dst_hbm.at[pl.ds(sid * ET, ET)], dstbuf)
        ones = jnp.full((16,), 1.0, jnp.float32)

        def step(i, _):
            idx = dstbuf[pl.ds(i * 16, 16)]
            plsc.addupdate_scatter(degbuf, [idx], ones)
            return 0

        lax.fori_loop(0, ET // 16, step, 0)
        pltpu.sync_copy(degbuf, degsh.at[sid])

    plsc.subcore_barrier()

    @pl.when(cid == 0)
    def _():
        r0 = sid * W16

        def z2(k, _):
            dinvb[pl.ds(k * 16, 16)] = jnp.zeros((16,), jnp.float32)
            return 0

        lax.fori_loop(0, W16 // 16, z2, 0)
        for s in range(NS):
            pltpu.sync_copy(degsh.at[s, pl.ds(r0, W16)], tmpb)

            def acc(k, _):
                sl = pl.ds(k * 16, 16)
                dinvb[sl] = dinvb[sl] + tmpb[sl]
                return 0

            lax.fori_loop(0, W16 // 16, acc, 0)

        def newton(k, _):
            sl = pl.ds(k * 16, 16)
            dinvb[sl] = _rsqrt16(dinvb[sl] + 1.0)
            return 0

        lax.fori_loop(0, W16 // 16, newton, 0)

        def rep(r, _):
            splat = plsc.load_gather(dinvb, [jnp.full((16,), 0, jnp.int32) + r])
            for k in range(D // 16):
                repb[r, pl.ds(k * 16, 16)] = splat
            return 0

        lax.fori_loop(0, W16, rep, 0)
        pltpu.sync_copy(repb, dinv_hbm.at[pl.ds(r0, W16)])


@functools.cache
def _sc_mesh():
    return plsc.VectorSubcoreMesh(
        core_axis_name="c", subcore_axis_name="s", num_cores=NC, num_subcores=NS
    )


@functools.cache
def _deg_call():
    return pl.kernel(
        _deg_body,
        out_type=jax.ShapeDtypeStruct((NP, D), jnp.float32),
        mesh=_sc_mesh(),
        compiler_params=pltpu.CompilerParams(
            needs_layout_passes=False, use_tc_tiling_on_sc=False
        ),
        scratch_types=[
            pltpu.VMEM((ET,), jnp.int32),
            pltpu.VMEM((NP,), jnp.float32),
            pltpu.VMEM((W16,), jnp.float32),
            pltpu.VMEM((W16,), jnp.float32),
            pltpu.VMEM((W16, D), jnp.float32),
            pltpu.VMEM_SHARED((NS, NP), jnp.float32),
        ],
    )


RD = 8
NGROUP = NCHUNK // RD


def _prop_body(hs_hbm, src_hbm, dst_hbm, out_hbm, acc, srcb, dstb, rows,
               *sems):
    gsem = sems[:RD]
    ssem = sems[RD:2 * RD]
    isem = sems[2 * RD:]
    cid = lax.axis_index("c")
    sid = lax.axis_index("s")
    wid = sid * NC + cid
    r0 = sid * RT
    pltpu.sync_copy(src_hbm.at[wid, pl.ds(0, RD)], srcb.at[0])
    pltpu.sync_copy(dst_hbm.at[wid, pl.ds(0, RD)], dstb.at[0])
    pltpu.async_copy(src_hbm.at[wid, pl.ds(RD, RD)], srcb.at[1], isem[0])
    pltpu.async_copy(dst_hbm.at[wid, pl.ds(RD, RD)], dstb.at[1], isem[1])
    pltpu.sync_copy(hs_hbm.at[pl.ds(r0, RT)], acc.at[pl.ds(r0, RT)])
    plsc.subcore_barrier()

    for b in range(RD):
        pltpu.async_copy(hs_hbm.at[srcb.at[0, b]], rows.at[b], gsem[b])

    def group(gi, _):
        par = lax.rem(gi, 2)
        nxt = 1 - par
        for b in range(RD):
            pltpu.make_async_copy(hs_hbm.at[srcb.at[par, b]], rows.at[b],
                                  gsem[b]).wait()
            pltpu.async_copy(rows.at[b], acc.at[dstb.at[par, b]], ssem[b],
                             add=True)
        pltpu.make_async_copy(src_hbm.at[wid, pl.ds(0, RD)], srcb.at[nxt],
                              isem[0]).wait()
        pltpu.make_async_copy(dst_hbm.at[wid, pl.ds(0, RD)], dstb.at[nxt],
                              isem[1]).wait()
        for b in range(RD):
            pltpu.make_async_copy(rows.at[b], acc.at[dstb.at[par, b]],
                                  ssem[b]).wait()
            pltpu.async_copy(hs_hbm.at[srcb.at[nxt, b]], rows.at[b], gsem[b])
        off = (gi + 2) * RD
        safe = jnp.minimum(off, NCHUNK - RD)
        pltpu.async_copy(src_hbm.at[wid, pl.ds(safe, RD)], srcb.at[par],
                         isem[0])
        pltpu.async_copy(dst_hbm.at[wid, pl.ds(safe, RD)], dstb.at[par],
                         isem[1])
        return 0

    lax.fori_loop(0, NGROUP - 1, group, 0)

    par = (NGROUP - 1) % 2
    pltpu.make_async_copy(src_hbm.at[wid, pl.ds(0, RD)], srcb.at[1 - par],
                          isem[0]).wait()
    pltpu.make_async_copy(dst_hbm.at[wid, pl.ds(0, RD)], dstb.at[1 - par],
                          isem[1]).wait()
    for b in range(RD):
        pltpu.make_async_copy(hs_hbm.at[srcb.at[par, b]], rows.at[b],
                              gsem[b]).wait()
        pltpu.async_copy(rows.at[b], acc.at[dstb.at[par, b]], ssem[b],
                         add=True)
    for b in range(RD):
        pltpu.make_async_copy(rows.at[b], acc.at[dstb.at[par, b]],
                              ssem[b]).wait()
    plsc.subcore_barrier()

    pltpu.sync_copy(acc.at[pl.ds(r0, RT)], out_hbm.at[cid, pl.ds(r0, RT)])


@functools.cache
def _prop_call():
    return pl.kernel(
        _prop_body,
        out_type=jax.ShapeDtypeStruct((NC, N, D), jnp.float32),
        mesh=_sc_mesh(),
        compiler_params=pltpu.CompilerParams(
            needs_layout_passes=False, use_tc_tiling_on_sc=False
        ),
        scratch_types=[
            pltpu.VMEM_SHARED((N + 8, D), jnp.float32),
            pltpu.VMEM((2, RD, C), jnp.int32),
            pltpu.VMEM((2, RD, C), jnp.int32),
            pltpu.VMEM((RD, C, D), jnp.float32),
        ] + [pltpu.SemaphoreType.DMA] * (2 * RD + 2),
    )


_BN = 1000


def _k1_body(x_ref, dinv_ref, w1_ref, hs_ref):
    h = jnp.dot(x_ref[...], w1_ref[...], preferred_element_type=jnp.float32)
    hs_ref[...] = h * dinv_ref[...]


def _k2_body(p_ref, hs_ref, dinv_ref, b_ref, w2_ref, out_ref):
    s = p_ref[0] + p_ref[1] - hs_ref[...]
    h = jnp.maximum(s * dinv_ref[...] + b_ref[...], 0.0)
    h2 = jnp.dot(h, w2_ref[...], preferred_element_type=jnp.float32)
    out_ref[...] = h2 * dinv_ref[...]


def _k3_body(p_ref, hs_ref, dinv_ref, b_ref, onehot_ref, wc_ref, bc_ref,
             out_ref, s_acc, c_acc):
    i = pl.program_id(0)

    @pl.when(i == 0)
    def _():
        s_acc[...] = jnp.zeros_like(s_acc)
        c_acc[...] = jnp.zeros_like(c_acc)

    s = p_ref[0] + p_ref[1] - hs_ref[...]
    h = jnp.maximum(s * dinv_ref[...] + b_ref[...], 0.0)
    onehot = onehot_ref[...]
    tdot = lambda a, b: lax.dot_general(
        a, b, (((0,), (0,)), ((), ())), preferred_element_type=jnp.float32)
    s_acc[...] += tdot(onehot, h)
    c_acc[...] += tdot(onehot, jnp.ones_like(h))

    @pl.when(i == pl.num_programs(0) - 1)
    def _():
        pooled = s_acc[...] / jnp.maximum(c_acc[...], 1.0)
        out_ref[...] = (
            jnp.dot(pooled, wc_ref[...], preferred_element_type=jnp.float32)
            + bc_ref[...]
        )


def _row_spec(width):
    return pl.BlockSpec((_BN, width), lambda i: (i, 0))


def _full_spec(shape):
    return pl.BlockSpec(shape, lambda i: tuple(0 for _ in shape))


_k1_call = pl.pallas_call(
    _k1_body,
    grid=(N // _BN,),
    in_specs=[
        _row_spec(D),
        _row_spec(D),
        _full_spec((D, D)),
    ],
    out_specs=_row_spec(D),
    out_shape=jax.ShapeDtypeStruct((N, D), jnp.float32),
)

_k2_call = pl.pallas_call(
    _k2_body,
    grid=(N // _BN,),
    in_specs=[
        pl.BlockSpec((NC, _BN, D), lambda i: (0, i, 0)),
        _row_spec(D),
        _row_spec(D),
        _full_spec((1, D)),
        _full_spec((D, D)),
    ],
    out_specs=_row_spec(D),
    out_shape=jax.ShapeDtypeStruct((N, D), jnp.float32),
)

_k3_call = pl.pallas_call(
    _k3_body,
    grid=(N // _BN,),
    in_specs=[
        pl.BlockSpec((NC, _BN, D), lambda i: (0, i, 0)),
        _row_spec(D),
        _row_spec(D),
        _full_spec((1, D)),
        _row_spec(G),
        _full_spec((D, OUT)),
        _full_spec((1, OUT)),
    ],
    out_specs=_full_spec((G, OUT)),
    out_shape=jax.ShapeDtypeStruct((G, OUT), jnp.float32),
    scratch_shapes=[
        pltpu.VMEM((G, D), jnp.float32),
        pltpu.VMEM((G, D), jnp.float32),
    ],
)


def kernel(x, edge_index, batch, W1, b1, W2, b2, Wc, bc):
    ei = edge_index.astype(jnp.int32)
    src = ei[0]
    dst = ei[1]
    dinvrep = _deg_call()(dst)
    h1s = _k1_call(x, dinvrep, W1)
    npad = NW * EWP - E
    padi = jnp.arange(npad, dtype=jnp.int32)
    src3 = jnp.concatenate([src, padi % N])
    src3 = src3.reshape(NW, NCHUNK, C)
    dst3 = jnp.concatenate([dst, N + (padi % 8)])
    dst3 = dst3.reshape(NW, NCHUNK, C)
    p1 = _prop_call()(h1s, src3, dst3)
    h2s = _k2_call(p1, h1s, dinvrep, b1.reshape(1, D), W2)
    p2 = _prop_call()(h2s, src3, dst3)
    onehot = (batch.astype(jnp.int32)[:, None]
              == jnp.arange(G, dtype=jnp.int32)).astype(jnp.float32)
    return _k3_call(
        p2, h2s, dinvrep, b2.reshape(1, D), onehot, Wc, bc.reshape(1, OUT),
    )


# speedup vs baseline: 1.0157x; 1.0157x over previous
import functools

import jax
import jax.numpy as jnp
from jax import lax
from jax.experimental import pallas as pl
from jax.experimental.pallas import tpu as pltpu
from jax.experimental.pallas import tpu_sc as plsc

N = 10000
D = 128
E = 320000
G = 64
OUT = 16

NC = 2
NS = 16
NW = NC * NS
EW = E // NW
C = 64
EWP = 10240
NCHUNK = EWP // C
RT = N // NS
RC = 25
NRC = RT // RC

NP = 10240
ET = E // NS
W16 = NP // NS


def _rsqrt16(d):
    i = plsc.bitcast(d, jnp.int32)
    y = plsc.bitcast(jnp.int32(0x5F3759DF) - (i >> 1), jnp.float32)
    for _ in range(3):
        y = y * (1.5 - 0.5 * d * y * y)
    return y


def _deg_body(dst_hbm, dinv_hbm, dstbuf, degbuf, tmpb, dinvb, repb, degsh):
    cid = lax.axis_index("c")
    sid = lax.axis_index("s")

    @pl.when(cid == 0)
    def _():
        def z(i, _):
            degbuf[pl.ds(i * 16, 16)] = jnp.zeros((16,), jnp.float32)
            return 0

        lax.fori_loop(0, NP // 16, z, 0)
        pltpu.sync_copy(dst_hbm.at[pl.ds(sid * ET, ET)], dstbuf)
        ones = jnp.full((16,), 1.0, jnp.float32)

        def step(i, _):
            idx = dstbuf[pl.ds(i * 16, 16)]
            plsc.addupdate_scatter(degbuf, [idx], ones)
            return 0

        lax.fori_loop(0, ET // 16, step, 0)
        pltpu.sync_copy(degbuf, degsh.at[sid])

    plsc.subcore_barrier()

    @pl.when(cid == 0)
    def _():
        r0 = sid * W16

        def z2(k, _):
            dinvb[pl.ds(k * 16, 16)] = jnp.zeros((16,), jnp.float32)
            return 0

        lax.fori_loop(0, W16 // 16, z2, 0)
        for s in range(NS):
            pltpu.sync_copy(degsh.at[s, pl.ds(r0, W16)], tmpb)

            def acc(k, _):
                sl = pl.ds(k * 16, 16)
                dinvb[sl] = dinvb[sl] + tmpb[sl]
                return 0

            lax.fori_loop(0, W16 // 16, acc, 0)

        def newton(k, _):
            sl = pl.ds(k * 16, 16)
            dinvb[sl] = _rsqrt16(dinvb[sl] + 1.0)
            return 0

        lax.fori_loop(0, W16 // 16, newton, 0)

        def rep(r, _):
            splat = plsc.load_gather(dinvb, [jnp.full((16,), 0, jnp.int32) + r])
            for k in range(D // 16):
                repb[r, pl.ds(k * 16, 16)] = splat
            return 0

        lax.fori_loop(0, W16, rep, 0)
        pltpu.sync_copy(repb, dinv_hbm.at[pl.ds(r0, W16)])


@functools.cache
def _sc_mesh():
    return plsc.VectorSubcoreMesh(
        core_axis_name="c", subcore_axis_name="s", num_cores=NC, num_subcores=NS
    )


@functools.cache
def _deg_call():
    return pl.kernel(
        _deg_body,
        out_type=jax.ShapeDtypeStruct((NP, D), jnp.float32),
        mesh=_sc_mesh(),
        compiler_params=pltpu.CompilerParams(
            needs_layout_passes=False, use_tc_tiling_on_sc=False
        ),
        scratch_types=[
            pltpu.VMEM((ET,), jnp.int32),
            pltpu.VMEM((NP,), jnp.float32),
            pltpu.VMEM((W16,), jnp.float32),
            pltpu.VMEM((W16,), jnp.float32),
            pltpu.VMEM((W16, D), jnp.float32),
            pltpu.VMEM_SHARED((NS, NP), jnp.float32),
        ],
    )


RD = 5
NGROUP = NCHUNK // RD


def _prop_body(hs_hbm, src_hbm, dst_hbm, out_hbm, acc, srcb, dstb, rows,
               *sems):
    gsem = sems[:RD]
    ssem = sems[RD:2 * RD]
    isem = sems[2 * RD:]
    cid = lax.axis_index("c")
    sid = lax.axis_index("s")
    wid = sid * NC + cid
    r0 = sid * RT
    pltpu.sync_copy(src_hbm.at[wid, pl.ds(0, RD)], srcb.at[0])
    pltpu.sync_copy(dst_hbm.at[wid, pl.ds(0, RD)], dstb.at[0])
    pltpu.async_copy(src_hbm.at[wid, pl.ds(RD, RD)], srcb.at[1], isem[0])
    pltpu.async_copy(dst_hbm.at[wid, pl.ds(RD, RD)], dstb.at[1], isem[1])
    pltpu.sync_copy(hs_hbm.at[pl.ds(r0, RT)], acc.at[pl.ds(r0, RT)])
    plsc.subcore_barrier()

    for b in range(RD):
        pltpu.async_copy(hs_hbm.at[srcb.at[0, b]], rows.at[b], gsem[b])

    def group(gi, _):
        par = lax.rem(gi, 2)
        nxt = 1 - par
        for b in range(RD):
            pltpu.make_async_copy(hs_hbm.at[srcb.at[par, b]], rows.at[b],
                                  gsem[b]).wait()
            pltpu.async_copy(rows.at[b], acc.at[dstb.at[par, b]], ssem[b],
                             add=True)
        pltpu.make_async_copy(src_hbm.at[wid, pl.ds(0, RD)], srcb.at[nxt],
                              isem[0]).wait()
        pltpu.make_async_copy(dst_hbm.at[wid, pl.ds(0, RD)], dstb.at[nxt],
                              isem[1]).wait()
        for b in range(RD):
            pltpu.make_async_copy(rows.at[b], acc.at[dstb.at[par, b]],
                                  ssem[b]).wait()
            pltpu.async_copy(hs_hbm.at[srcb.at[nxt, b]], rows.at[b], gsem[b])
        off = (gi + 2) * RD
        safe = jnp.minimum(off, NCHUNK - RD)
        pltpu.async_copy(src_hbm.at[wid, pl.ds(safe, RD)], srcb.at[par],
                         isem[0])
        pltpu.async_copy(dst_hbm.at[wid, pl.ds(safe, RD)], dstb.at[par],
                         isem[1])
        return 0

    lax.fori_loop(0, NGROUP - 1, group, 0)

    par = (NGROUP - 1) % 2
    pltpu.make_async_copy(src_hbm.at[wid, pl.ds(0, RD)], srcb.at[1 - par],
                          isem[0]).wait()
    pltpu.make_async_copy(dst_hbm.at[wid, pl.ds(0, RD)], dstb.at[1 - par],
                          isem[1]).wait()
    for b in range(RD):
        pltpu.make_async_copy(hs_hbm.at[srcb.at[par, b]], rows.at[b],
                              gsem[b]).wait()
        pltpu.async_copy(rows.at[b], acc.at[dstb.at[par, b]], ssem[b],
                         add=True)
    for b in range(RD):
        pltpu.make_async_copy(rows.at[b], acc.at[dstb.at[par, b]],
                              ssem[b]).wait()
    plsc.subcore_barrier()

    pltpu.sync_copy(acc.at[pl.ds(r0, RT)], out_hbm.at[cid, pl.ds(r0, RT)])


@functools.cache
def _prop_call():
    return pl.kernel(
        _prop_body,
        out_type=jax.ShapeDtypeStruct((NC, N, D), jnp.float32),
        mesh=_sc_mesh(),
        compiler_params=pltpu.CompilerParams(
            needs_layout_passes=False, use_tc_tiling_on_sc=False
        ),
        scratch_types=[
            pltpu.VMEM_SHARED((N + 8, D), jnp.float32),
            pltpu.VMEM((2, RD, C), jnp.int32),
            pltpu.VMEM((2, RD, C), jnp.int32),
            pltpu.VMEM((RD, C, D), jnp.float32),
        ] + [pltpu.SemaphoreType.DMA] * (2 * RD + 2),
    )


_BN = 1000


def _k1_body(x_ref, dinv_ref, w1_ref, hs_ref):
    h = jnp.dot(x_ref[...], w1_ref[...], preferred_element_type=jnp.float32)
    hs_ref[...] = h * dinv_ref[...]


def _k2_body(p_ref, hs_ref, dinv_ref, b_ref, w2_ref, out_ref):
    s = p_ref[0] + p_ref[1] - hs_ref[...]
    h = jnp.maximum(s * dinv_ref[...] + b_ref[...], 0.0)
    h2 = jnp.dot(h, w2_ref[...], preferred_element_type=jnp.float32)
    out_ref[...] = h2 * dinv_ref[...]


def _k3_body(p_ref, hs_ref, dinv_ref, b_ref, onehot_ref, wc_ref, bc_ref,
             out_ref, s_acc, c_acc):
    i = pl.program_id(0)

    @pl.when(i == 0)
    def _():
        s_acc[...] = jnp.zeros_like(s_acc)
        c_acc[...] = jnp.zeros_like(c_acc)

    s = p_ref[0] + p_ref[1] - hs_ref[...]
    h = jnp.maximum(s * dinv_ref[...] + b_ref[...], 0.0)
    onehot = onehot_ref[...]
    tdot = lambda a, b: lax.dot_general(
        a, b, (((0,), (0,)), ((), ())), preferred_element_type=jnp.float32)
    s_acc[...] += tdot(onehot, h)
    c_acc[...] += tdot(onehot, jnp.ones_like(h))

    @pl.when(i == pl.num_programs(0) - 1)
    def _():
        pooled = s_acc[...] / jnp.maximum(c_acc[...], 1.0)
        out_ref[...] = (
            jnp.dot(pooled, wc_ref[...], preferred_element_type=jnp.float32)
            + bc_ref[...]
        )


def _row_spec(width):
    return pl.BlockSpec((_BN, width), lambda i: (i, 0))


def _full_spec(shape):
    return pl.BlockSpec(shape, lambda i: tuple(0 for _ in shape))


_k1_call = pl.pallas_call(
    _k1_body,
    grid=(N // _BN,),
    in_specs=[
        _row_spec(D),
        _row_spec(D),
        _full_spec((D, D)),
    ],
    out_specs=_row_spec(D),
    out_shape=jax.ShapeDtypeStruct((N, D), jnp.float32),
)

_k2_call = pl.pallas_call(
    _k2_body,
    grid=(N // _BN,),
    in_specs=[
        pl.BlockSpec((NC, _BN, D), lambda i: (0, i, 0)),
        _row_spec(D),
        _row_spec(D),
        _full_spec((1, D)),
        _full_spec((D, D)),
    ],
    out_specs=_row_spec(D),
    out_shape=jax.ShapeDtypeStruct((N, D), jnp.float32),
)

_k3_call = pl.pallas_call(
    _k3_body,
    grid=(N // _BN,),
    in_specs=[
        pl.BlockSpec((NC, _BN, D), lambda i: (0, i, 0)),
        _row_spec(D),
        _row_spec(D),
        _full_spec((1, D)),
        _row_spec(G),
        _full_spec((D, OUT)),
        _full_spec((1, OUT)),
    ],
    out_specs=_full_spec((G, OUT)),
    out_shape=jax.ShapeDtypeStruct((G, OUT), jnp.float32),
    scratch_shapes=[
        pltpu.VMEM((G, D), jnp.float32),
        pltpu.VMEM((G, D), jnp.float32),
    ],
)


def kernel(x, edge_index, batch, W1, b1, W2, b2, Wc, bc):
    ei = edge_index.astype(jnp.int32)
    src = ei[0]
    dst = ei[1]
    dinvrep = _deg_call()(dst)
    h1s = _k1_call(x, dinvrep, W1)
    npad = NW * EWP - E
    padi = jnp.arange(npad, dtype=jnp.int32)
    src3 = jnp.concatenate([src, padi % N])
    src3 = src3.reshape(NW, NCHUNK, C)
    dst3 = jnp.concatenate([dst, N + (padi % 8)])
    dst3 = dst3.reshape(NW, NCHUNK, C)
    p1 = _prop_call()(h1s, src3, dst3)
    h2s = _k2_call(p1, h1s, dinvrep, b1.reshape(1, D), W2)
    p2 = _prop_call()(h2s, src3, dst3)
    onehot = (batch.astype(jnp.int32)[:, None]
              == jnp.arange(G, dtype=jnp.int32)).astype(jnp.float32)
    return _k3_call(
        p2, h2s, dinvrep, b2.reshape(1, D), onehot, Wc, bc.reshape(1, OUT),
    )


# speedup vs baseline: 1.0226x; 1.0068x over previous
import functools

import jax
import jax.numpy as jnp
from jax import lax
from jax.experimental import pallas as pl
from jax.experimental.pallas import tpu as pltpu
from jax.experimental.pallas import tpu_sc as plsc

N = 10000
D = 128
E = 320000
G = 64
OUT = 16

NC = 2
NS = 16
NW = NC * NS
EW = E // NW
C = 64
EWP = 10240
NCHUNK = EWP // C
RT = N // NS
RC = 25
NRC = RT // RC

NP = 10240
ET = E // NS
W16 = NP // NS


def _rsqrt16(d):
    i = plsc.bitcast(d, jnp.int32)
    y = plsc.bitcast(jnp.int32(0x5F3759DF) - (i >> 1), jnp.float32)
    for _ in range(3):
        y = y * (1.5 - 0.5 * d * y * y)
    return y


def _deg_body(dst_hbm, dinv_hbm, dstbuf, degbuf, tmpb, dinvb, repb, degsh):
    cid = lax.axis_index("c")
    sid = lax.axis_index("s")

    @pl.when(cid == 0)
    def _():
        def z(i, _):
            for u in range(8):
                degbuf[pl.ds(i * 128 + u * 16, 16)] = jnp.zeros(
                    (16,), jnp.float32)
            return 0

        lax.fori_loop(0, NP // 128, z, 0)
        pltpu.sync_copy(dst_hbm.at[pl.ds(sid * ET, ET)], dstbuf)
        ones = jnp.full((16,), 1.0, jnp.float32)

        def step(i, _):
            for u in range(8):
                idx = dstbuf[pl.ds(i * 128 + u * 16, 16)]
                plsc.addupdate_scatter(degbuf, [idx], ones)
            return 0

        lax.fori_loop(0, ET // 128, step, 0)
        pltpu.sync_copy(degbuf, degsh.at[sid])

    plsc.subcore_barrier()

    @pl.when(cid == 0)
    def _():
        r0 = sid * W16

        def z2(k, _):
            dinvb[pl.ds(k * 16, 16)] = jnp.zeros((16,), jnp.float32)
            return 0

        lax.fori_loop(0, W16 // 16, z2, 0)
        for s in range(NS):
            pltpu.sync_copy(degsh.at[s, pl.ds(r0, W16)], tmpb)

            def acc(k, _):
                sl = pl.ds(k * 16, 16)
                dinvb[sl] = dinvb[sl] + tmpb[sl]
                return 0

            lax.fori_loop(0, W16 // 16, acc, 0)

        def newton(k, _):
            sl = pl.ds(k * 16, 16)
            dinvb[sl] = _rsqrt16(dinvb[sl] + 1.0)
            return 0

        lax.fori_loop(0, W16 // 16, newton, 0)

        def rep(r, _):
            for u in range(4):
                rr = r * 4 + u
                splat = plsc.load_gather(
                    dinvb, [jnp.full((16,), 0, jnp.int32) + rr])
                for k in range(D // 16):
                    repb[rr, pl.ds(k * 16, 16)] = splat
            return 0

        lax.fori_loop(0, W16 // 4, rep, 0)
        pltpu.sync_copy(repb, dinv_hbm.at[pl.ds(r0, W16)])


@functools.cache
def _sc_mesh():
    return plsc.VectorSubcoreMesh(
        core_axis_name="c", subcore_axis_name="s", num_cores=NC, num_subcores=NS
    )


@functools.cache
def _deg_call():
    return pl.kernel(
        _deg_body,
        out_type=jax.ShapeDtypeStruct((NP, D), jnp.float32),
        mesh=_sc_mesh(),
        compiler_params=pltpu.CompilerParams(
            needs_layout_passes=False, use_tc_tiling_on_sc=False
        ),
        scratch_types=[
            pltpu.VMEM((ET,), jnp.int32),
            pltpu.VMEM((NP,), jnp.float32),
            pltpu.VMEM((W16,), jnp.float32),
            pltpu.VMEM((W16,), jnp.float32),
            pltpu.VMEM((W16, D), jnp.float32),
            pltpu.VMEM_SHARED((NS, NP), jnp.float32),
        ],
    )


RD = 5
NGROUP = NCHUNK // RD


def _prop_body(hs_hbm, src_hbm, dst_hbm, out_hbm, acc, srcb, dstb, rows,
               *sems):
    gsem = sems[:RD]
    ssem = sems[RD:2 * RD]
    isem = sems[2 * RD:]
    cid = lax.axis_index("c")
    sid = lax.axis_index("s")
    wid = sid * NC + cid
    r0 = sid * RT
    pltpu.sync_copy(src_hbm.at[wid, pl.ds(0, RD)], srcb.at[0])
    pltpu.sync_copy(dst_hbm.at[wid, pl.ds(0, RD)], dstb.at[0])
    pltpu.async_copy(src_hbm.at[wid, pl.ds(RD, RD)], srcb.at[1], isem[0])
    pltpu.async_copy(dst_hbm.at[wid, pl.ds(RD, RD)], dstb.at[1], isem[1])
    pltpu.sync_copy(hs_hbm.at[pl.ds(r0, RT)], acc.at[pl.ds(r0, RT)])
    plsc.subcore_barrier()

    for b in range(RD):
        pltpu.async_copy(hs_hbm.at[srcb.at[0, b]], rows.at[b], gsem[b])

    def group(gi, _):
        par = lax.rem(gi, 2)
        nxt = 1 - par
        for b in range(RD):
            pltpu.make_async_copy(hs_hbm.at[srcb.at[par, b]], rows.at[b],
                                  gsem[b]).wait()
            pltpu.async_copy(rows.at[b], acc.at[dstb.at[par, b]], ssem[b],
                             add=True)
        pltpu.make_async_copy(src_hbm.at[wid, pl.ds(0, RD)], srcb.at[nxt],
                              isem[0]).wait()
        pltpu.make_async_copy(dst_hbm.at[wid, pl.ds(0, RD)], dstb.at[nxt],
                              isem[1]).wait()
        for b in range(RD):
            pltpu.make_async_copy(rows.at[b], acc.at[dstb.at[par, b]],
                                  ssem[b]).wait()
            pltpu.async_copy(hs_hbm.at[srcb.at[nxt, b]], rows.at[b], gsem[b])
        off = (gi + 2) * RD
        safe = jnp.minimum(off, NCHUNK - RD)
        pltpu.async_copy(src_hbm.at[wid, pl.ds(safe, RD)], srcb.at[par],
                         isem[0])
        pltpu.async_copy(dst_hbm.at[wid, pl.ds(safe, RD)], dstb.at[par],
                         isem[1])
        return 0

    lax.fori_loop(0, NGROUP - 1, group, 0)

    par = (NGROUP - 1) % 2
    pltpu.make_async_copy(src_hbm.at[wid, pl.ds(0, RD)], srcb.at[1 - par],
                          isem[0]).wait()
    pltpu.make_async_copy(dst_hbm.at[wid, pl.ds(0, RD)], dstb.at[1 - par],
                          isem[1]).wait()
    for b in range(RD):
        pltpu.make_async_copy(hs_hbm.at[srcb.at[par, b]], rows.at[b],
                              gsem[b]).wait()
        pltpu.async_copy(rows.at[b], acc.at[dstb.at[par, b]], ssem[b],
                         add=True)
    for b in range(RD):
        pltpu.make_async_copy(rows.at[b], acc.at[dstb.at[par, b]],
                              ssem[b]).wait()
    plsc.subcore_barrier()

    pltpu.sync_copy(acc.at[pl.ds(r0, RT)], out_hbm.at[cid, pl.ds(r0, RT)])


@functools.cache
def _prop_call():
    return pl.kernel(
        _prop_body,
        out_type=jax.ShapeDtypeStruct((NC, N, D), jnp.float32),
        mesh=_sc_mesh(),
        compiler_params=pltpu.CompilerParams(
            needs_layout_passes=False, use_tc_tiling_on_sc=False
        ),
        scratch_types=[
            pltpu.VMEM_SHARED((N + 8, D), jnp.float32),
            pltpu.VMEM((2, RD, C), jnp.int32),
            pltpu.VMEM((2, RD, C), jnp.int32),
            pltpu.VMEM((RD, C, D), jnp.float32),
        ] + [pltpu.SemaphoreType.DMA] * (2 * RD + 2),
    )


_BN = 1000


def _k1_body(x_ref, dinv_ref, w1_ref, hs_ref):
    h = jnp.dot(x_ref[...], w1_ref[...], preferred_element_type=jnp.float32)
    hs_ref[...] = h * dinv_ref[...]


def _k2_body(p_ref, hs_ref, dinv_ref, b_ref, w2_ref, out_ref):
    s = p_ref[0] + p_ref[1] - hs_ref[...]
    h = jnp.maximum(s * dinv_ref[...] + b_ref[...], 0.0)
    h2 = jnp.dot(h, w2_ref[...], preferred_element_type=jnp.float32)
    out_ref[...] = h2 * dinv_ref[...]


def _k3_body(p_ref, hs_ref, dinv_ref, b_ref, onehot_ref, wc_ref, bc_ref,
             out_ref, s_acc, c_acc):
    i = pl.program_id(0)

    @pl.when(i == 0)
    def _():
        s_acc[...] = jnp.zeros_like(s_acc)
        c_acc[...] = jnp.zeros_like(c_acc)

    s = p_ref[0] + p_ref[1] - hs_ref[...]
    h = jnp.maximum(s * dinv_ref[...] + b_ref[...], 0.0)
    onehot = onehot_ref[...]
    tdot = lambda a, b: lax.dot_general(
        a, b, (((0,), (0,)), ((), ())), preferred_element_type=jnp.float32)
    s_acc[...] += tdot(onehot, h)
    c_acc[...] += tdot(onehot, jnp.ones_like(h))

    @pl.when(i == pl.num_programs(0) - 1)
    def _():
        pooled = s_acc[...] / jnp.maximum(c_acc[...], 1.0)
        out_ref[...] = (
            jnp.dot(pooled, wc_ref[...], preferred_element_type=jnp.float32)
            + bc_ref[...]
        )


def _row_spec(width):
    return pl.BlockSpec((_BN, width), lambda i: (i, 0))


def _full_spec(shape):
    return pl.BlockSpec(shape, lambda i: tuple(0 for _ in shape))


_k1_call = pl.pallas_call(
    _k1_body,
    grid=(N // _BN,),
    in_specs=[
        _row_spec(D),
        _row_spec(D),
        _full_spec((D, D)),
    ],
    out_specs=_row_spec(D),
    out_shape=jax.ShapeDtypeStruct((N, D), jnp.float32),
)

_k2_call = pl.pallas_call(
    _k2_body,
    grid=(N // _BN,),
    in_specs=[
        pl.BlockSpec((NC, _BN, D), lambda i: (0, i, 0)),
        _row_spec(D),
        _row_spec(D),
        _full_spec((1, D)),
        _full_spec((D, D)),
    ],
    out_specs=_row_spec(D),
    out_shape=jax.ShapeDtypeStruct((N, D), jnp.float32),
)

_k3_call = pl.pallas_call(
    _k3_body,
    grid=(N // _BN,),
    in_specs=[
        pl.BlockSpec((NC, _BN, D), lambda i: (0, i, 0)),
        _row_spec(D),
        _row_spec(D),
        _full_spec((1, D)),
        _row_spec(G),
        _full_spec((D, OUT)),
        _full_spec((1, OUT)),
    ],
    out_specs=_full_spec((G, OUT)),
    out_shape=jax.ShapeDtypeStruct((G, OUT), jnp.float32),
    scratch_shapes=[
        pltpu.VMEM((G, D), jnp.float32),
        pltpu.VMEM((G, D), jnp.float32),
    ],
)


def kernel(x, edge_index, batch, W1, b1, W2, b2, Wc, bc):
    ei = edge_index.astype(jnp.int32)
    src = ei[0]
    dst = ei[1]
    dinvrep = _deg_call()(dst)
    h1s = _k1_call(x, dinvrep, W1)
    npad = NW * EWP - E
    padi = jnp.arange(npad, dtype=jnp.int32)
    src3 = jnp.concatenate([src, padi % N])
    src3 = src3.reshape(NW, NCHUNK, C)
    dst3 = jnp.concatenate([dst, N + (padi % 8)])
    dst3 = dst3.reshape(NW, NCHUNK, C)
    p1 = _prop_call()(h1s, src3, dst3)
    h2s = _k2_call(p1, h1s, dinvrep, b1.reshape(1, D), W2)
    p2 = _prop_call()(h2s, src3, dst3)
    onehot = (batch.astype(jnp.int32)[:, None]
              == jnp.arange(G, dtype=jnp.int32)).astype(jnp.float32)
    return _k3_call(
        p2, h2s, dinvrep, b2.reshape(1, D), onehot, Wc, bc.reshape(1, OUT),
    )
